# baseline (speedup 1.0000x reference)
import math

import jax
import jax.numpy as jnp
from jax.experimental import pallas as pl
from jax.experimental.pallas import tpu as pltpu

D_H = 2048
E = 64
NB = 16
R = D_H // NB
SCALE = 1.0 / math.sqrt(2048.0)


def _body(ctx_ref, wq_ref, wk_ref, keys_ref, out_ref, t_ref):
    i = pl.program_id(0)

    @pl.when(i == 0)
    def _init():
        t_ref[...] = jnp.zeros_like(t_ref)

    q_blk = jax.lax.dot_general(
        ctx_ref[...], wq_ref[...], (((1,), (1,)), ((), ())),
        preferred_element_type=jnp.float32)
    t_ref[...] += jax.lax.dot_general(
        q_blk, wk_ref[...], (((1,), (0,)), ((), ())),
        preferred_element_type=jnp.float32)

    @pl.when(i == NB - 1)
    def _fin():
        t = t_ref[...]
        s = jax.lax.dot_general(
            t, keys_ref[...], (((1,), (1,)), ((), ())),
            preferred_element_type=jnp.float32) * SCALE
        m = jnp.max(s, axis=-1, keepdims=True)
        ex = jnp.exp(s - m)
        out_ref[...] = ex / jnp.sum(ex, axis=-1, keepdims=True)


def kernel(expert_outputs, context, keys, Wq, Wk):
    del expert_outputs
    ctx2 = context.reshape(1, D_H)
    gate = pl.pallas_call(
        _body,
        grid=(NB,),
        in_specs=[
            pl.BlockSpec((1, D_H), lambda i: (0, 0)),
            pl.BlockSpec((R, D_H), lambda i: (i, 0)),
            pl.BlockSpec((R, D_H), lambda i: (i, 0)),
            pl.BlockSpec((E, D_H), lambda i: (0, 0)),
        ],
        out_specs=pl.BlockSpec((1, E), lambda i: (0, 0)),
        out_shape=jax.ShapeDtypeStruct((1, E), jnp.float32),
        scratch_shapes=[pltpu.VMEM((1, D_H), jnp.float32)],
        compiler_params=pltpu.CompilerParams(
            dimension_semantics=("arbitrary",),
        ),
    )(ctx2, Wq, Wk, keys)
    return gate.reshape(E)


# speedup vs baseline: 1.3718x; 1.3718x over previous
import math

import jax
import jax.numpy as jnp
from jax.experimental import pallas as pl
from jax.experimental.pallas import tpu as pltpu

D_H = 2048
E = 64
NB = 4
R = D_H // NB
SCALE = 1.0 / math.sqrt(2048.0)


def _body(ctx_ref, wq_ref, wk_ref, keys_ref, out_ref, t_ref):
    i = pl.program_id(0)

    @pl.when(i == 0)
    def _init():
        t_ref[...] = jnp.zeros_like(t_ref)

    q_blk = jax.lax.dot_general(
        ctx_ref[...], wq_ref[...], (((1,), (1,)), ((), ())),
        preferred_element_type=jnp.float32)
    t_ref[...] += jax.lax.dot_general(
        q_blk, wk_ref[...], (((1,), (0,)), ((), ())),
        preferred_element_type=jnp.float32)

    @pl.when(i == NB - 1)
    def _fin():
        t = t_ref[...]
        s = jax.lax.dot_general(
            t, keys_ref[...], (((1,), (1,)), ((), ())),
            preferred_element_type=jnp.float32) * SCALE
        m = jnp.max(s, axis=-1, keepdims=True)
        ex = jnp.exp(s - m)
        out_ref[...] = ex / jnp.sum(ex, axis=-1, keepdims=True)


def kernel(expert_outputs, context, keys, Wq, Wk):
    del expert_outputs
    ctx2 = context.reshape(1, D_H)
    gate = pl.pallas_call(
        _body,
        grid=(NB,),
        in_specs=[
            pl.BlockSpec((1, D_H), lambda i: (0, 0)),
            pl.BlockSpec((R, D_H), lambda i: (i, 0)),
            pl.BlockSpec((R, D_H), lambda i: (i, 0)),
            pl.BlockSpec((E, D_H), lambda i: (0, 0)),
        ],
        out_specs=pl.BlockSpec((1, E), lambda i: (0, 0)),
        out_shape=jax.ShapeDtypeStruct((1, E), jnp.float32),
        scratch_shapes=[pltpu.VMEM((1, D_H), jnp.float32)],
        compiler_params=pltpu.CompilerParams(
            dimension_semantics=("arbitrary",),
        ),
    )(ctx2, Wq, Wk, keys)
    return gate.reshape(E)
